# baseline (speedup 1.0000x reference)
import jax
import jax.numpy as jnp
from jax import lax
from jax.experimental import pallas as pl
from jax.experimental.pallas import tpu as pltpu
from jax.experimental.pallas import tpu_sc as plsc

SENT_LEN = 50
BATCH = 1024
MAX_WORD_LEN = 20
CHAR_VOCAB = 262
CHAR_EMBED = 20
DPAD = 24
N_WORDS = SENT_LEN * BATCH
PADW = 24
TABN = CHAR_VOCAB * CHAR_EMBED

NUM_CORES = 2
NUM_SUBCORES = 16
NW = NUM_CORES * NUM_SUBCORES
SW = N_WORDS // NW
C = 80
NCH = SW // C
NWG = C // 16


def _sc_body(inp_ref, w2c_ref, tab_ref, out_e_ref, out_l_ref,
             widx, spell, lens, tab_v, ebuf, sem_s):
  cid = lax.axis_index("c")
  sid = lax.axis_index("s")
  wid = sid * NUM_CORES + cid
  wbase = wid * SW

  pltpu.sync_copy(tab_ref, tab_v)

  def chunk_body(ci, carry):
    cb = wbase + ci * C
    pltpu.sync_copy(inp_ref.at[pl.ds(cb, C)], widx)
    pltpu.async_copy(w2c_ref.at[widx], spell, sem_s).wait()

    def extract_len(i, carry2):
      lane = lax.iota(jnp.int32, 16)
      wj = i * 16 + lane
      col_len = jnp.full((16,), MAX_WORD_LEN, jnp.int32)
      lv = plsc.load_gather(spell, [wj, col_len])
      lens[pl.ds(ci * C + i * 16, 16)] = lv
      return carry2

    lax.fori_loop(0, NWG, extract_len, 0)

    def build_cg(cg, carry2):
      c = cg // NWG
      g = cg - c * NWG
      lane = lax.iota(jnp.int32, 16)
      wl = g * 16 + lane
      c_vec = jnp.full((16,), 0, jnp.int32) + c
      chars = plsc.load_gather(spell, [wl, c_vec])
      fb = chars * CHAR_EMBED

      def build_d(d, carry3):
        vals = plsc.load_gather(tab_v, [fb + d])
        ebuf[c, d, pl.ds(g * 16, 16)] = vals
        return carry3

      lax.fori_loop(0, CHAR_EMBED, build_d, 0, unroll=10)
      return carry2

    lax.fori_loop(0, MAX_WORD_LEN * NWG, build_cg, 0, unroll=2)

    pltpu.sync_copy(ebuf, out_e_ref.at[:, :, pl.ds(cb, C)])
    return carry

  lax.fori_loop(0, NCH, chunk_body, 0)
  pltpu.sync_copy(lens, out_l_ref.at[pl.ds(wbase, SW)])


@jax.jit
def _char2vec(inp_flat, w2c_pad, tab_flat):
  mesh = plsc.VectorSubcoreMesh(
      core_axis_name="c", subcore_axis_name="s",
      num_cores=NUM_CORES, num_subcores=NUM_SUBCORES)
  k = pl.kernel(
      _sc_body,
      out_type=[
          jax.ShapeDtypeStruct((MAX_WORD_LEN, DPAD, N_WORDS), jnp.float32),
          jax.ShapeDtypeStruct((N_WORDS,), jnp.int32),
      ],
      mesh=mesh,
      scratch_types=[
          pltpu.VMEM((C,), jnp.int32),
          pltpu.VMEM((C, PADW), jnp.int32),
          pltpu.VMEM((SW,), jnp.int32),
          pltpu.VMEM((TABN,), jnp.float32),
          pltpu.VMEM((MAX_WORD_LEN, DPAD, C), jnp.float32),
          pltpu.SemaphoreType.DMA,
      ],
      compiler_params=pltpu.CompilerParams(
          use_tc_tiling_on_sc=False, needs_layout_passes=False),
  )
  return k(inp_flat, w2c_pad, tab_flat)


def kernel(inp, word2chars, charEmbedTable):
  sent_len, batch, _ = inp.shape
  inp_flat = inp.reshape(-1)
  w2c_pad = jnp.pad(word2chars, ((0, 0), (0, PADW - MAX_WORD_LEN - 1)))
  tab_flat = charEmbedTable.reshape(-1)
  emb_t, len_flat = _char2vec(inp_flat, w2c_pad, tab_flat)
  char_embeds = jnp.transpose(emb_t[:, :CHAR_EMBED, :], (2, 0, 1))
  return (char_embeds, len_flat)


# speedup vs baseline: 1.0374x; 1.0374x over previous
import jax
import jax.numpy as jnp
from jax import lax
from jax.experimental import pallas as pl
from jax.experimental.pallas import tpu as pltpu
from jax.experimental.pallas import tpu_sc as plsc

SENT_LEN = 50
BATCH = 1024
MAX_WORD_LEN = 20
CHAR_VOCAB = 262
CHAR_EMBED = 20
DPAD = 24
N_WORDS = SENT_LEN * BATCH
WORD_VOCAB = 100000
SROW = 24
NSROW = WORD_VOCAB * (MAX_WORD_LEN + 1) // SROW
TABN = CHAR_VOCAB * CHAR_EMBED

NUM_CORES = 2
NUM_SUBCORES = 16
NW = NUM_CORES * NUM_SUBCORES
SW = N_WORDS // NW
C = 80
NCH = SW // C
NWG = C // 16


def _sc_body(inp_ref, w2c_ref, tab_ref, out_e_ref, out_l_ref,
             widx, ridx, offv, spell, lens, tab_v, ebuf, sem_s):
  cid = lax.axis_index("c")
  sid = lax.axis_index("s")
  wid = sid * NUM_CORES + cid
  wbase = wid * SW

  pltpu.sync_copy(tab_ref, tab_v)

  def chunk_body(ci, carry):
    cb = wbase + ci * C
    pltpu.sync_copy(inp_ref.at[pl.ds(cb, C)], widx)

    def rowidx(i, carry2):
      wv = widx[pl.ds(i * 16, 16)]
      t21 = wv * (MAX_WORD_LEN + 1)
      r0 = lax.shift_right_logical(wv * 7, 3)
      ridx[0, pl.ds(i * 16, 16)] = r0
      ridx[1, pl.ds(i * 16, 16)] = jnp.minimum(r0 + 1, NSROW - 1)
      offv[pl.ds(i * 16, 16)] = t21 - r0 * SROW
      return carry2

    lax.fori_loop(0, NWG, rowidx, 0)

    cp0 = pltpu.async_copy(w2c_ref.at[ridx.at[0]], spell.at[pl.ds(0, C)],
                           sem_s)
    cp1 = pltpu.async_copy(w2c_ref.at[ridx.at[1]], spell.at[pl.ds(C, C)],
                           sem_s)
    cp0.wait()
    cp1.wait()

    def extract_len(i, carry2):
      lane = lax.iota(jnp.int32, 16)
      wl = i * 16 + lane
      oj = plsc.load_gather(offv, [wl]) + MAX_WORD_LEN
      k = lax.shift_right_logical(oj * 2731, 16)
      m = oj - k * SROW
      lv = plsc.load_gather(spell, [k * C + wl, m])
      lens[pl.ds(ci * C + i * 16, 16)] = lv
      return carry2

    lax.fori_loop(0, NWG, extract_len, 0)

    def build_cg(cg, carry2):
      c = cg // NWG
      g = cg - c * NWG
      lane = lax.iota(jnp.int32, 16)
      wl = g * 16 + lane
      oj = plsc.load_gather(offv, [wl]) + c
      k = lax.shift_right_logical(oj * 2731, 16)
      m = oj - k * SROW
      chars = plsc.load_gather(spell, [k * C + wl, m])
      fb = chars * CHAR_EMBED

      def build_d(d, carry3):
        vals = plsc.load_gather(tab_v, [fb + d])
        ebuf[c, d, pl.ds(g * 16, 16)] = vals
        return carry3

      lax.fori_loop(0, CHAR_EMBED, build_d, 0, unroll=5)
      return carry2

    lax.fori_loop(0, MAX_WORD_LEN * NWG, build_cg, 0)

    pltpu.sync_copy(ebuf, out_e_ref.at[:, :, pl.ds(cb, C)])
    return carry

  lax.fori_loop(0, NCH, chunk_body, 0)
  pltpu.sync_copy(lens, out_l_ref.at[pl.ds(wbase, SW)])


@jax.jit
def _char2vec(inp_flat, w2c_rows, tab_flat):
  mesh = plsc.VectorSubcoreMesh(
      core_axis_name="c", subcore_axis_name="s",
      num_cores=NUM_CORES, num_subcores=NUM_SUBCORES)
  k = pl.kernel(
      _sc_body,
      out_type=[
          jax.ShapeDtypeStruct((MAX_WORD_LEN, DPAD, N_WORDS), jnp.float32),
          jax.ShapeDtypeStruct((N_WORDS,), jnp.int32),
      ],
      mesh=mesh,
      scratch_types=[
          pltpu.VMEM((C,), jnp.int32),
          pltpu.VMEM((2, C), jnp.int32),
          pltpu.VMEM((C,), jnp.int32),
          pltpu.VMEM((2 * C, SROW), jnp.int32),
          pltpu.VMEM((SW,), jnp.int32),
          pltpu.VMEM((TABN,), jnp.float32),
          pltpu.VMEM((MAX_WORD_LEN, DPAD, C), jnp.float32),
          pltpu.SemaphoreType.DMA,
      ],
      compiler_params=pltpu.CompilerParams(
          use_tc_tiling_on_sc=False, needs_layout_passes=False),
  )
  return k(inp_flat, w2c_rows, tab_flat)


def kernel(inp, word2chars, charEmbedTable):
  sent_len, batch, _ = inp.shape
  inp_flat = inp.reshape(-1)
  w2c_rows = word2chars.reshape(NSROW, SROW)
  tab_flat = charEmbedTable.reshape(-1)
  emb_t, len_flat = _char2vec(inp_flat, w2c_rows, tab_flat)
  char_embeds = jnp.transpose(emb_t[:, :CHAR_EMBED, :], (2, 0, 1))
  return (char_embeds, len_flat)


# speedup vs baseline: 1.0910x; 1.0517x over previous
import jax
import jax.numpy as jnp
from jax import lax
from jax.experimental import pallas as pl
from jax.experimental.pallas import tpu as pltpu
from jax.experimental.pallas import tpu_sc as plsc

SENT_LEN = 50
BATCH = 1024
MAX_WORD_LEN = 20
CHAR_VOCAB = 262
CHAR_EMBED = 20
DPAD = 24
N_WORDS = SENT_LEN * BATCH
WORD_VOCAB = 100000
SROW = 24
NSROW = WORD_VOCAB * (MAX_WORD_LEN + 1) // SROW
TABN = CHAR_VOCAB * CHAR_EMBED

NUM_CORES = 2
NUM_SUBCORES = 16
NW = NUM_CORES * NUM_SUBCORES
SW = N_WORDS // NW
C = 80
NCH = SW // C
NWG = C // 16


def _sc_body(inp_ref, w2c_ref, tab_ref, out_e_ref, out_l_ref,
             widx, ridx, offv, spell, lens, tab_v, ebuf, sem_s, sem_w):
  cid = lax.axis_index("c")
  sid = lax.axis_index("s")
  wid = sid * NUM_CORES + cid
  wbase = wid * SW

  pltpu.sync_copy(tab_ref, tab_v)

  def chunk_body(ci, carry):
    cb = wbase + ci * C
    buf = lax.rem(ci, 2)
    pltpu.sync_copy(inp_ref.at[pl.ds(cb, C)], widx)

    def rowidx(i, carry2):
      wv = widx[pl.ds(i * 16, 16)]
      t21 = wv * (MAX_WORD_LEN + 1)
      r0 = lax.shift_right_logical(wv * 7, 3)
      ridx[0, pl.ds(i * 16, 16)] = r0
      ridx[1, pl.ds(i * 16, 16)] = jnp.minimum(r0 + 1, NSROW - 1)
      offv[pl.ds(i * 16, 16)] = t21 - r0 * SROW
      return carry2

    lax.fori_loop(0, NWG, rowidx, 0)

    cp0 = pltpu.async_copy(w2c_ref.at[ridx.at[0]], spell.at[pl.ds(0, C)],
                           sem_s)
    cp1 = pltpu.async_copy(w2c_ref.at[ridx.at[1]], spell.at[pl.ds(C, C)],
                           sem_s)
    cp0.wait()
    cp1.wait()

    def extract_len(i, carry2):
      lane = lax.iota(jnp.int32, 16)
      wl = i * 16 + lane
      oj = plsc.load_gather(offv, [wl]) + MAX_WORD_LEN
      k = lax.shift_right_logical(oj * 2731, 16)
      m = oj - k * SROW
      lv = plsc.load_gather(spell, [k * C + wl, m])
      lens[pl.ds(ci * C + i * 16, 16)] = lv
      return carry2

    lax.fori_loop(0, NWG, extract_len, 0)

    def build_cg(cg, carry2):
      c = cg // NWG
      g = cg - c * NWG
      lane = lax.iota(jnp.int32, 16)
      wl = g * 16 + lane
      oj = plsc.load_gather(offv, [wl]) + c
      k = lax.shift_right_logical(oj * 2731, 16)
      m = oj - k * SROW
      chars = plsc.load_gather(spell, [k * C + wl, m])
      fb = chars * CHAR_EMBED

      def build_d(d, carry3):
        vals = plsc.load_gather(tab_v, [fb + d])
        ebuf[buf, c, d, pl.ds(g * 16, 16)] = vals
        return carry3

      lax.fori_loop(0, CHAR_EMBED, build_d, 0, unroll=5)
      return carry2

    lax.fori_loop(0, MAX_WORD_LEN * NWG, build_cg, 0)

    @pl.when(ci >= 2)
    def _():
      pltpu.make_async_copy(
          out_e_ref.at[:, :, pl.ds(wbase, C)], ebuf.at[buf], sem_w).wait()

    pltpu.async_copy(ebuf.at[buf], out_e_ref.at[:, :, pl.ds(cb, C)], sem_w)
    return carry

  lax.fori_loop(0, NCH, chunk_body, 0)
  for _ in range(2):
    pltpu.make_async_copy(
        out_e_ref.at[:, :, pl.ds(wbase, C)], ebuf.at[0], sem_w).wait()
  pltpu.sync_copy(lens, out_l_ref.at[pl.ds(wbase, SW)])


@jax.jit
def _char2vec(inp_flat, w2c_rows, tab_flat):
  mesh = plsc.VectorSubcoreMesh(
      core_axis_name="c", subcore_axis_name="s",
      num_cores=NUM_CORES, num_subcores=NUM_SUBCORES)
  k = pl.kernel(
      _sc_body,
      out_type=[
          jax.ShapeDtypeStruct((MAX_WORD_LEN, DPAD, N_WORDS), jnp.float32),
          jax.ShapeDtypeStruct((N_WORDS,), jnp.int32),
      ],
      mesh=mesh,
      scratch_types=[
          pltpu.VMEM((C,), jnp.int32),
          pltpu.VMEM((2, C), jnp.int32),
          pltpu.VMEM((C,), jnp.int32),
          pltpu.VMEM((2 * C, SROW), jnp.int32),
          pltpu.VMEM((SW,), jnp.int32),
          pltpu.VMEM((TABN,), jnp.float32),
          pltpu.VMEM((2, MAX_WORD_LEN, DPAD, C), jnp.float32),
          pltpu.SemaphoreType.DMA,
          pltpu.SemaphoreType.DMA,
      ],
      compiler_params=pltpu.CompilerParams(
          use_tc_tiling_on_sc=False, needs_layout_passes=False),
  )
  return k(inp_flat, w2c_rows, tab_flat)


def kernel(inp, word2chars, charEmbedTable):
  sent_len, batch, _ = inp.shape
  inp_flat = inp.reshape(-1)
  w2c_rows = word2chars.reshape(NSROW, SROW)
  tab_flat = charEmbedTable.reshape(-1)
  emb_t, len_flat = _char2vec(inp_flat, w2c_rows, tab_flat)
  char_embeds = jnp.transpose(emb_t[:, :CHAR_EMBED, :], (2, 0, 1))
  return (char_embeds, len_flat)
